# baseline (speedup 1.0000x reference)
import jax
import jax.numpy as jnp
from jax import lax
from jax.experimental import pallas as pl
from jax.experimental.pallas import tpu as pltpu
from jax.experimental.pallas import tpu_sc as plsc

PAD = 1
B = 4
T = 2048
D = 1024
NROWS = 2050
NC = 2
NS = 16
L = 16
NW = NC * NS
TOK_PER_W = (B * T) // NW
SEG_PER_ROW = T // TOK_PER_W
CHUNK = 32
NCHUNK = TOK_PER_W // CHUNK
VREGS_PER_SEG = TOK_PER_W // L
NBUF = 3


def _body(x_hbm, tablef_hbm, outf_hbm, xrow_ref, xseg_ref, xshared_ref, *rest):
    bufs = rest[:NBUF]
    gsems = rest[NBUF:2 * NBUF]
    ssems = rest[2 * NBUF:3 * NBUF]

    sid = lax.axis_index("s")
    wid = sid * NC + lax.axis_index("c")
    row = wid // SEG_PER_ROW
    seg = wid % SEG_PER_ROW

    pltpu.sync_copy(x_hbm.at[row], xrow_ref)

    def acc_body(j, acc_v):
        v = xrow_ref[pl.ds(j * L, L)]
        return acc_v + (v != PAD).astype(jnp.int32)

    acc_v = lax.fori_loop(
        0, seg * VREGS_PER_SEG, acc_body, jnp.zeros((L,), jnp.int32)
    )
    c0 = jnp.sum(acc_v)

    seg_v = jnp.zeros((L,), jnp.int32)
    for k in range(VREGS_PER_SEG):
        v = xrow_ref[pl.ds((seg * VREGS_PER_SEG + k) * L, L)]
        seg_v = seg_v + (v != PAD).astype(jnp.int32)
    pad_free = jnp.sum(seg_v) == TOK_PER_W

    base = wid * TOK_PER_W

    def ring(gather_chunk):
        handles_g = [None] * NBUF
        handles_s = [None] * NBUF

        def scatter_chunk(d, db):
            return pltpu.async_copy(
                bufs[db],
                outf_hbm.at[pl.ds((base + d * CHUNK) * D, CHUNK * D)],
                ssems[db],
            )

        for c in range(NCHUNK):
            b = c % NBUF
            if handles_s[b] is not None:
                handles_s[b].wait()
            handles_g[b] = gather_chunk(c, b)
            d = c - (NBUF - 1)
            if d >= 0:
                db = d % NBUF
                for h in handles_g[db]:
                    h.wait()
                handles_s[db] = scatter_chunk(d, db)
        for d in range(max(0, NCHUNK - NBUF + 1), NCHUNK):
            db = d % NBUF
            for h in handles_g[db]:
                h.wait()
            handles_s[db] = scatter_chunk(d, db)
        for b in range(NBUF):
            if handles_s[b] is not None:
                handles_s[b].wait()

    def fast_path():
        g0 = pl.multiple_of((c0 + 2) * D, 8)

        def gather_chunk(c, b):
            return [
                pltpu.async_copy(
                    tablef_hbm.at[pl.ds(g0 + c * (CHUNK * D), CHUNK * D)],
                    bufs[b],
                    gsems[b],
                )
            ]

        ring(gather_chunk)

    def slow_path():
        carries = [c0]
        for t in range(TOK_PER_W):
            m = (xseg_ref[t] != PAD).astype(jnp.int32)
            carries.append(carries[t] + m)
        positions = [
            jnp.where(
                carries[t + 1] > carries[t], carries[t + 1] + 1, jnp.int32(PAD)
            )
            for t in range(TOK_PER_W)
        ]

        def gather_chunk(c, b):
            handles = []
            for j in range(CHUNK):
                p = positions[c * CHUNK + j]
                handles.append(
                    pltpu.async_copy(
                        tablef_hbm.at[pl.ds(pl.multiple_of(p * D, 8), D)],
                        bufs[b].at[pl.ds(j * D, D)],
                        gsems[b],
                    )
                )
            return handles

        ring(gather_chunk)

    fast_path()


_lookup = pl.kernel(
    _body,
    out_type=jax.ShapeDtypeStruct((B * T * D,), jnp.float32),
    mesh=plsc.VectorSubcoreMesh(
        core_axis_name="c", subcore_axis_name="s", num_cores=NC, num_subcores=NS
    ),
    scratch_types=(
        [
            pltpu.VMEM((T,), jnp.int32),
            pltpu.SMEM((TOK_PER_W,), jnp.int32),
            pltpu.VMEM_SHARED((NS, TOK_PER_W), jnp.int32),
        ]
        + [pltpu.VMEM((CHUNK * D,), jnp.float32) for _ in range(NBUF)]
        + [pltpu.SemaphoreType.DMA for _ in range(2 * NBUF)]
    ),
    compiler_params=pltpu.CompilerParams(needs_layout_passes=False),
)


def kernel(x, table):
    out = _lookup(x, table.reshape(-1))
    return out.reshape(B, T, D)


# speedup vs baseline: 1.8867x; 1.8867x over previous
import jax
import jax.numpy as jnp
from jax import lax
from jax.experimental import pallas as pl
from jax.experimental.pallas import tpu as pltpu
from jax.experimental.pallas import tpu_sc as plsc

PAD = 1
B = 4
T = 2048
D = 1024
NC = 2
NS = 16
L = 16
NW = NC * NS
TOK_PER_W = (B * T) // NW
SEG_PER_ROW = T // TOK_PER_W
CHUNK = 16
NCHUNK = TOK_PER_W // CHUNK
VREGS_PER_SEG = TOK_PER_W // L
NBUF = 6


def _body(x_hbm, table_hbm, out_hbm, xrow_ref, *rest):
    idxs = rest[:NCHUNK]
    bufs = rest[NCHUNK:NCHUNK + NBUF]
    gsems = rest[NCHUNK + NBUF:NCHUNK + 2 * NBUF]
    ssems = rest[NCHUNK + 2 * NBUF:NCHUNK + 3 * NBUF]

    wid = lax.axis_index("s") * NC + lax.axis_index("c")
    row = wid // SEG_PER_ROW
    seg = wid % SEG_PER_ROW

    pltpu.sync_copy(x_hbm.at[row], xrow_ref)

    def acc_body(j, acc):
        v = xrow_ref[pl.ds(j * L, L)]
        return acc + jnp.sum((v != PAD).astype(jnp.int32))

    carry = lax.fori_loop(0, seg * VREGS_PER_SEG, acc_body, jnp.int32(0))

    for k in range(VREGS_PER_SEG):
        i = seg * VREGS_PER_SEG + k
        v = xrow_ref[pl.ds(i * L, L)]
        m = (v != PAD).astype(jnp.int32)
        pos = (jnp.cumsum(m) + carry) * m + PAD
        idxs[(k * L) // CHUNK][pl.ds((k * L) % CHUNK, L)] = pos
        carry = carry + jnp.sum(m)

    base = wid * TOK_PER_W
    handles_g = [None] * NBUF
    handles_s = [None] * NBUF
    for c in range(NBUF - 1):
        b = c % NBUF
        handles_g[b] = pltpu.async_copy(table_hbm.at[idxs[c]], bufs[b], gsems[b])
    for c in range(NCHUNK):
        b = c % NBUF
        nc = c + NBUF - 1
        if nc < NCHUNK:
            nb = nc % NBUF
            if handles_s[nb] is not None:
                handles_s[nb].wait()
            handles_g[nb] = pltpu.async_copy(
                table_hbm.at[idxs[nc]], bufs[nb], gsems[nb]
            )
        handles_g[b].wait()
        handles_s[b] = pltpu.async_copy(
            bufs[b], out_hbm.at[pl.ds(base + c * CHUNK, CHUNK)], ssems[b]
        )
    for b in range(NBUF):
        handles_s[b].wait()


_lookup = pl.kernel(
    _body,
    out_type=jax.ShapeDtypeStruct((B * T, D), jnp.float32),
    mesh=plsc.VectorSubcoreMesh(
        core_axis_name="c", subcore_axis_name="s", num_cores=NC, num_subcores=NS
    ),
    scratch_types=(
        [pltpu.VMEM((T,), jnp.int32)]
        + [pltpu.VMEM((CHUNK,), jnp.int32) for _ in range(NCHUNK)]
        + [pltpu.VMEM((CHUNK, D), jnp.float32) for _ in range(NBUF)]
        + [pltpu.SemaphoreType.DMA for _ in range(2 * NBUF)]
    ),
    compiler_params=pltpu.CompilerParams(needs_layout_passes=False),
)


def kernel(x, table):
    out = _lookup(x, table)
    return out.reshape(B, T, D)
